# Initial kernel scaffold; baseline (speedup 1.0000x reference)
import jax
import jax.numpy as jnp
from jax.experimental import pallas as pl


def kernel(x_numerical, x_categorical_idx, x_categorical_meta, W_num, b_num, table, W_meta, b_meta, feature_identity, gamma, beta):
    raise NotImplementedError("write your pallas kernel here")


# speedup vs baseline: 1.6856x; 1.6856x over previous
import functools

import jax
import jax.numpy as jnp
import numpy as np
from jax import lax
from jax.experimental import pallas as pl
from jax.experimental.pallas import tpu as pltpu
from jax.experimental.pallas import tpu_sc as plsc

B, N_NUM, N_CAT, D, V = 16384, 13, 26, 32, 1000000
NF = N_NUM + N_CAT
TOT = B * N_CAT
NW = 32
PER_W = TOT // NW
CH = 128
K = 8
GROUPS = PER_W // (CH * K)
BL = 512


def _sc_gather(table, idx_flat):
    idx3 = idx_flat.reshape(NW, PER_W // CH, CH)
    mesh = plsc.VectorSubcoreMesh(core_axis_name="c", subcore_axis_name="s")

    @functools.partial(
        pl.kernel,
        mesh=mesh,
        out_type=jax.ShapeDtypeStruct((TOT, D), jnp.float32),
        scratch_types=[
            pltpu.VMEM((PER_W // CH, CH), jnp.int32),
            pltpu.VMEM((K * CH, D), jnp.float32),
            pltpu.SemaphoreType.DMA,
        ],
        compiler_params=pltpu.CompilerParams(use_tc_tiling_on_sc=False),
    )
    def gather_kernel(table_hbm, idx_hbm, out_hbm, idx_v, rows_v, sem):
        wid = lax.axis_index("s") * 2 + lax.axis_index("c")
        pltpu.sync_copy(idx_hbm.at[wid], idx_v)
        base = wid * PER_W

        def group(g, carry):
            handles = []
            for b in range(K):
                handles.append(
                    pltpu.async_copy(
                        table_hbm.at[idx_v.at[g * K + b]],
                        rows_v.at[pl.ds(b * CH, CH)],
                        sem,
                    )
                )
            for h in handles:
                h.wait()
            pltpu.sync_copy(rows_v, out_hbm.at[pl.ds(base + g * (K * CH), K * CH)])
            return carry

        lax.fori_loop(0, GROUPS, group, 0)

    return gather_kernel(table, idx3)


def _tc_body(xT_r, idxT_r, metaT_r, embT_r, WnT_r, WmT_r, bn_r, bm_r,
             fid_r, g_r, bt_r, h0_o, raw_o, mask_o, um_o):
    f32 = jnp.float32
    gelu = lambda t: 0.5 * t * (1.0 + lax.erf(t * np.float32(0.7071067811865476)))

    x3 = xT_r[...].reshape(3, N_NUM, BL)
    m3 = metaT_r[...].reshape(N_CAT, 2, BL)
    e3 = embT_r[...].reshape(N_CAT, D, BL)
    idxv = idxT_r[...]
    fid = fid_r[...]
    gam = g_r[...][None]
    bet = bt_r[...][None]

    wcol = lambda ref, j: ref[:, j:j + 1][None]

    zn = (x3[0][:, None, :] * wcol(WnT_r, 0)
          + x3[1][:, None, :] * wcol(WnT_r, 1)
          + x3[2][:, None, :] * wcol(WnT_r, 2)
          + bn_r[...][None])
    zn = gelu(zn) + fid[:N_NUM]

    zc = (e3
          + m3[:, 0:1, :] * wcol(WmT_r, 0)
          + m3[:, 1:2, :] * wcol(WmT_r, 1)
          + bm_r[...][None])
    zc = gelu(zc) + fid[N_NUM:]

    def ln(z):
        mean = jnp.mean(z, axis=1, keepdims=True)
        var = jnp.mean(z * z, axis=1, keepdims=True) - mean * mean
        return (z - mean) * lax.rsqrt(var + np.float32(1e-5)) * gam + bet

    h0_o[:N_NUM * D, :] = ln(zn).reshape(N_NUM * D, BL)
    h0_o[N_NUM * D:, :] = ln(zc).reshape(N_CAT * D, BL)

    idx_f = idxv.astype(f32)
    raw_o[...] = jnp.concatenate([x3[0], idx_f], axis=0)

    mask_o[...] = jnp.where(
        lax.broadcasted_iota(jnp.int32, (NF, BL), 0) < N_NUM,
        np.float32(1.0), np.float32(0.0))

    um_num = jnp.concatenate(
        [x3[1][:, None, :], x3[2][:, None, :]], axis=1).reshape(2 * N_NUM, BL)
    sign = (idxv % 2 * 2 - 1).astype(f32)
    tf = 0.5 + sign * (0.5 - 0.5 * m3[:, 0, :])
    um_cat = jnp.concatenate(
        [tf[:, None, :], m3[:, 1, :][:, None, :]], axis=1).reshape(2 * N_CAT, BL)
    um_o[...] = jnp.concatenate([um_num, um_cat], axis=0)


def kernel(x_numerical, x_categorical_idx, x_categorical_meta, W_num, b_num,
           table, W_meta, b_meta, feature_identity, gamma, beta):
    f32 = jnp.float32
    idx = x_categorical_idx.astype(jnp.int32)

    xT = jnp.transpose(x_numerical, (2, 1, 0)).reshape(3 * N_NUM, B)
    idxT = jnp.transpose(idx, (1, 0))
    metaT = jnp.transpose(x_categorical_meta, (1, 2, 0)).reshape(2 * N_CAT, B)

    emb = _sc_gather(table, idxT.reshape(TOT))
    embT = jnp.transpose(emb.reshape(N_CAT, B, D), (0, 2, 1)).reshape(N_CAT * D, B)

    WnT = jnp.transpose(W_num, (1, 0))
    WmT = jnp.transpose(W_meta, (1, 0))
    bn = b_num.reshape(D, 1)
    bm = b_meta.reshape(D, 1)
    fid = feature_identity.reshape(NF, D, 1)
    gam = gamma.reshape(D, 1)
    bet = beta.reshape(D, 1)

    grid = (B // BL,)
    lane = lambda rows: pl.BlockSpec((rows, BL), lambda i: (0, i))
    full = lambda shp: pl.BlockSpec(shp, lambda i: tuple(0 for _ in shp))
    h0T, rawT, maskT, umT = pl.pallas_call(
        _tc_body,
        grid=grid,
        in_specs=[
            lane(3 * N_NUM), lane(N_CAT), lane(2 * N_CAT), lane(N_CAT * D),
            full(WnT.shape), full(WmT.shape), full(bn.shape), full(bm.shape),
            full(fid.shape), full(gam.shape), full(bet.shape),
        ],
        out_specs=[lane(NF * D), lane(NF), lane(NF), lane(2 * NF)],
        out_shape=[
            jax.ShapeDtypeStruct((NF * D, B), f32),
            jax.ShapeDtypeStruct((NF, B), f32),
            jax.ShapeDtypeStruct((NF, B), f32),
            jax.ShapeDtypeStruct((2 * NF, B), f32),
        ],
        compiler_params=pltpu.CompilerParams(
            dimension_semantics=("parallel",)),
    )(xT, idxT, metaT, embT, WnT, WmT, bn, bm, fid, gam, bet)

    h_0 = jnp.transpose(h0T.reshape(NF, D, B), (2, 0, 1))
    raw = jnp.transpose(rawT, (1, 0)).reshape(B, NF, 1)
    mask = jnp.transpose(maskT, (1, 0))
    um = jnp.transpose(umT.reshape(NF, 2, B), (2, 0, 1))
    return (h_0, raw, mask, um)
